# Initial kernel scaffold; baseline (speedup 1.0000x reference)
import functools

import jax
import jax.numpy as jnp
from jax import lax
from jax.experimental import pallas as pl
from jax.experimental.pallas import tpu as pltpu
from jax.experimental.pallas import tpu_sc as plsc

N = 10000
E = 320000
D = 128
G = 64
NC = 2
NS = 16
NW = NC * NS
EPW = E // NW
C = 80
NCHUNK = EPW // C
RPT = N // NS

_mesh = plsc.VectorSubcoreMesh(core_axis_name="c", subcore_axis_name="s")


def _lane_bcast(v, i):
    return jnp.take(v, jnp.full((16,), i, dtype=jnp.int32),
                    mode="promise_in_bounds")


@functools.partial(
    pl.kernel,
    mesh=_mesh,
    out_type=jax.ShapeDtypeStruct((NC, N, D), jnp.float32),
    scratch_types=[
        pltpu.VMEM((C,), jnp.int32),
        pltpu.VMEM((C,), jnp.int32),
        pltpu.VMEM((C,), jnp.float32),
        pltpu.VMEM((C, D), jnp.float32),
        pltpu.VMEM_SHARED((N, D), jnp.float32),
        pltpu.SemaphoreType.DMA,
    ],
)
def _spmm(h_hbm, src_hbm, dst_hbm, ea_hbm, out_hbm,
          src_v, dst_v, ea_v, rows_v, agg_sh, sem):
    c = lax.axis_index("c")
    s = lax.axis_index("s")
    wid = s * NC + c

    def _zero_row(i, carry):
        z = jnp.zeros((16,), jnp.float32)
        for f in range(D // 16):
            rows_v[i, pl.ds(f * 16, 16)] = z
        return carry

    lax.fori_loop(0, C, _zero_row, 0)
    for k in range(RPT // C):
        pltpu.sync_copy(rows_v, agg_sh.at[pl.ds(s * RPT + k * C, C)])
    rem = RPT % C
    if rem:
        pltpu.sync_copy(rows_v.at[pl.ds(0, rem)],
                        agg_sh.at[pl.ds(s * RPT + (RPT // C) * C, rem)])
    plsc.subcore_barrier()

    def _chunk(j, carry):
        base = wid * EPW + j * C
        pltpu.sync_copy(src_hbm.at[pl.ds(base, C)], src_v)
        pltpu.sync_copy(dst_hbm.at[pl.ds(base, C)], dst_v)
        pltpu.sync_copy(ea_hbm.at[pl.ds(base, C)], ea_v)
        pltpu.async_copy(h_hbm.at[src_v], rows_v, sem).wait()

        def _grp(g, inner):
            e16 = ea_v[pl.ds(g * 16, 16)]
            for i in range(16):
                r = g * 16 + i
                eb = _lane_bcast(e16, i)
                for f in range(D // 16):
                    rows_v[r, pl.ds(f * 16, 16)] = (
                        rows_v[r, pl.ds(f * 16, 16)] * eb)
            return inner

        lax.fori_loop(0, C // 16, _grp, 0)
        pltpu.sync_copy(rows_v, agg_sh.at[dst_v], add=True)
        return carry

    lax.fori_loop(0, NCHUNK, _chunk, 0)

    plsc.subcore_barrier()
    pltpu.sync_copy(agg_sh.at[pl.ds(s * RPT, RPT)],
                    out_hbm.at[c, pl.ds(s * RPT, RPT)])


def _dense_body(a0, a1, h, wr, b, wt, o):
    acc = a0[...] + a1[...]
    o[...] = (jnp.dot(acc, wr[...], preferred_element_type=jnp.float32)
              + b[...]
              + jnp.dot(h[...], wt[...], preferred_element_type=jnp.float32))


def _dense(agg, h, WrT, b, WtT):
    BN = 1000
    return pl.pallas_call(
        _dense_body,
        grid=(N // BN,),
        in_specs=[
            pl.BlockSpec((BN, D), lambda i: (i, 0)),
            pl.BlockSpec((BN, D), lambda i: (i, 0)),
            pl.BlockSpec((BN, D), lambda i: (i, 0)),
            pl.BlockSpec((D, D), lambda i: (0, 0)),
            pl.BlockSpec((1, D), lambda i: (0, 0)),
            pl.BlockSpec((D, D), lambda i: (0, 0)),
        ],
        out_specs=pl.BlockSpec((BN, D), lambda i: (i, 0)),
        out_shape=jax.ShapeDtypeStruct((N, D), jnp.float32),
    )(agg[0], agg[1], h, WrT, b, WtT)


def _readout_body(h, w1, b1, w2, b2, bt, o):
    t = jnp.maximum(
        jnp.dot(h[...], w1[...], preferred_element_type=jnp.float32) + b1[...],
        0.0)
    y = jnp.dot(t, w2[...], preferred_element_type=jnp.float32) + b2[...]
    gids = lax.broadcasted_iota(jnp.int32, (G, N), 0)
    m = (bt[...] == gids).astype(jnp.float32)
    sums = jnp.dot(m, y, preferred_element_type=jnp.float32)
    cnts = jnp.sum(m, axis=1, keepdims=True)
    o[...] = sums / jnp.maximum(cnts, 1.0)


def _readout(h, W1T, b1, W2T, b2, batch2d):
    return pl.pallas_call(
        _readout_body,
        out_shape=jax.ShapeDtypeStruct((G, 1), jnp.float32),
    )(h, W1T, b1, W2T, b2, batch2d)


def kernel(x, edge_index, edge_attr, batch,
           W_rel0, b_rel0, W_root0,
           W_rel1, b_rel1, W_root1,
           W_rel2, b_rel2, W_root2,
           W_ro1, b_ro1, W_ro2, b_ro2):
    src = edge_index[0]
    dst = edge_index[1]
    h = x
    for Wr, b, Wt in ((W_rel0, b_rel0, W_root0),
                      (W_rel1, b_rel1, W_root1),
                      (W_rel2, b_rel2, W_root2)):
        agg = _spmm(h, src, dst, edge_attr)
        h = _dense(agg, h, Wr.T, b.reshape(1, D), Wt.T)
    return _readout(h, W_ro1.T, b_ro1.reshape(1, D // 2),
                    W_ro2.T, b_ro2.reshape(1, 1), batch.reshape(1, N))


# speedup vs baseline: 3.7702x; 3.7702x over previous
import functools

import jax
import jax.numpy as jnp
from jax import lax
from jax.experimental import pallas as pl
from jax.experimental.pallas import tpu as pltpu
from jax.experimental.pallas import tpu_sc as plsc

N = 10000
E = 320000
D = 128
G = 64
NC = 2
NS = 16
NW = NC * NS
EPW = E // NW
C = 80
NCHUNK = EPW // C
NP = 10240
RPT = NP // NS

_mesh = plsc.VectorSubcoreMesh(core_axis_name="c", subcore_axis_name="s")


def _lane_bcast(v, i):
    idx = jnp.full((16, 1), i, dtype=jnp.int32)
    dnums = lax.GatherDimensionNumbers(
        offset_dims=(), collapsed_slice_dims=(0,), start_index_map=(0,))
    return lax.gather(v, idx, dnums, slice_sizes=(1,),
                      mode=lax.GatherScatterMode.PROMISE_IN_BOUNDS)


@functools.partial(
    pl.kernel,
    mesh=_mesh,
    out_type=jax.ShapeDtypeStruct((NC, NP, D), jnp.float32),
    scratch_types=[
        pltpu.VMEM((C,), jnp.int32),
        pltpu.VMEM((C,), jnp.int32),
        pltpu.VMEM((C,), jnp.float32),
        pltpu.VMEM((C, D), jnp.float32),
        pltpu.VMEM_SHARED((NP, D), jnp.float32),
        pltpu.SemaphoreType.DMA,
    ],
)
def _spmm(h_hbm, src_hbm, dst_hbm, ea_hbm, out_hbm,
          src_v, dst_v, ea_v, rows_v, agg_sh, sem):
    c = lax.axis_index("c")
    s = lax.axis_index("s")
    wid = s * NC + c

    def _zero_row(i, carry):
        z = jnp.zeros((16,), jnp.float32)
        for f in range(D // 16):
            rows_v[i, pl.ds(f * 16, 16)] = z
        return carry

    lax.fori_loop(0, C, _zero_row, 0)
    for k in range(RPT // C):
        pltpu.sync_copy(rows_v, agg_sh.at[pl.ds(s * RPT + k * C, C)])
    plsc.subcore_barrier()

    def _chunk(j, carry):
        base = wid * EPW + j * C
        pltpu.sync_copy(src_hbm.at[pl.ds(base, C)], src_v)
        pltpu.sync_copy(dst_hbm.at[pl.ds(base, C)], dst_v)
        pltpu.sync_copy(ea_hbm.at[pl.ds(base, C)], ea_v)
        pltpu.async_copy(h_hbm.at[src_v], rows_v, sem).wait()

        def _grp(g, inner):
            e16 = ea_v[pl.ds(g * 16, 16)]
            for i in range(16):
                r = g * 16 + i
                eb = _lane_bcast(e16, i)
                for f in range(D // 16):
                    rows_v[r, pl.ds(f * 16, 16)] = (
                        rows_v[r, pl.ds(f * 16, 16)] * eb)
            return inner

        lax.fori_loop(0, C // 16, _grp, 0)
        pltpu.sync_copy(rows_v, agg_sh.at[dst_v], add=True)
        return carry

    lax.fori_loop(0, NCHUNK, _chunk, 0)

    plsc.subcore_barrier()
    pltpu.sync_copy(agg_sh.at[pl.ds(s * RPT, RPT)],
                    out_hbm.at[c, pl.ds(s * RPT, RPT)])


def _dense_body(a0, a1, h, wr, b, wt, o):
    acc = a0[...] + a1[...]
    o[...] = (jnp.dot(acc, wr[...], preferred_element_type=jnp.float32)
              + b[...]
              + jnp.dot(h[...], wt[...], preferred_element_type=jnp.float32))


def _dense(agg, h, WrT, b, WtT):
    BN = 1000
    return pl.pallas_call(
        _dense_body,
        grid=(N // BN,),
        in_specs=[
            pl.BlockSpec((BN, D), lambda i: (i, 0)),
            pl.BlockSpec((BN, D), lambda i: (i, 0)),
            pl.BlockSpec((BN, D), lambda i: (i, 0)),
            pl.BlockSpec((D, D), lambda i: (0, 0)),
            pl.BlockSpec((1, D), lambda i: (0, 0)),
            pl.BlockSpec((D, D), lambda i: (0, 0)),
        ],
        out_specs=pl.BlockSpec((BN, D), lambda i: (i, 0)),
        out_shape=jax.ShapeDtypeStruct((N, D), jnp.float32),
    )(agg[0], agg[1], h, WrT, b, WtT)


def _readout_body(h, w1, b1, w2, b2, bt, o):
    t = jnp.maximum(
        jnp.dot(h[...], w1[...], preferred_element_type=jnp.float32) + b1[...],
        0.0)
    y = jnp.dot(t, w2[...], preferred_element_type=jnp.float32) + b2[...]
    gids = lax.broadcasted_iota(jnp.int32, (G, N), 0)
    m = (bt[...] == gids).astype(jnp.float32)
    sums = jnp.dot(m, y, preferred_element_type=jnp.float32)
    cnts = jnp.sum(m, axis=1, keepdims=True)
    o[...] = sums / jnp.maximum(cnts, 1.0)


def _readout(h, W1T, b1, W2T, b2, batch2d):
    return pl.pallas_call(
        _readout_body,
        out_shape=jax.ShapeDtypeStruct((G, 1), jnp.float32),
    )(h, W1T, b1, W2T, b2, batch2d)


def kernel(x, edge_index, edge_attr, batch,
           W_rel0, b_rel0, W_root0,
           W_rel1, b_rel1, W_root1,
           W_rel2, b_rel2, W_root2,
           W_ro1, b_ro1, W_ro2, b_ro2):
    src = edge_index[0]
    dst = edge_index[1]
    h = x
    for Wr, b, Wt in ((W_rel0, b_rel0, W_root0),
                      (W_rel1, b_rel1, W_root1),
                      (W_rel2, b_rel2, W_root2)):
        agg = _spmm(h, src, dst, edge_attr)
        h = _dense(agg[:, :N, :], h, Wr.T, b.reshape(1, D), Wt.T)
    return _readout(h, W_ro1.T, b_ro1.reshape(1, D // 2),
                    W_ro2.T, b_ro2.reshape(1, 1), batch.reshape(1, N))


# speedup vs baseline: 9.6367x; 2.5560x over previous
import functools

import jax
import jax.numpy as jnp
from jax import lax
from jax.experimental import pallas as pl
from jax.experimental.pallas import tpu as pltpu
from jax.experimental.pallas import tpu_sc as plsc

N = 10000
E = 320000
D = 128
G = 64
NC = 2
NS = 16
NW = NC * NS
EPW = E // NW
C = 80
NCHUNK = EPW // C
NP = 10240
RPT = NP // NS

_mesh = plsc.VectorSubcoreMesh(core_axis_name="c", subcore_axis_name="s")


def _lane_bcast(v, i):
    idx = jnp.full((16, 1), i, dtype=jnp.int32)
    dnums = lax.GatherDimensionNumbers(
        offset_dims=(), collapsed_slice_dims=(0,), start_index_map=(0,))
    return lax.gather(v, idx, dnums, slice_sizes=(1,),
                      mode=lax.GatherScatterMode.PROMISE_IN_BOUNDS)


@functools.partial(
    pl.kernel,
    mesh=_mesh,
    out_type=jax.ShapeDtypeStruct((NC, NP, D), jnp.float32),
    scratch_types=[
        pltpu.VMEM((EPW,), jnp.int32),
        pltpu.VMEM((EPW,), jnp.int32),
        pltpu.VMEM((C,), jnp.int32),
        pltpu.VMEM((C,), jnp.float32),
        pltpu.VMEM((C,), jnp.float32),
        pltpu.VMEM((C, D), jnp.float32),
        pltpu.VMEM((C, D), jnp.float32),
        pltpu.VMEM_SHARED((NP, D), jnp.float32),
        pltpu.SemaphoreType.DMA,
        pltpu.SemaphoreType.DMA,
    ],
)
def _spmm(h_hbm, src_hbm, dst_hbm, ea_hbm, out_hbm,
          src_all, dst_all, dst_v, ea0, ea1, rows0, rows1, agg_sh,
          sem0, sem1):
    c = lax.axis_index("c")
    s = lax.axis_index("s")
    wid = s * NC + c
    ebase = wid * EPW

    pltpu.sync_copy(src_hbm.at[pl.ds(ebase, EPW)], src_all)
    pltpu.sync_copy(dst_hbm.at[pl.ds(ebase, EPW)], dst_all)

    def _zero_row(i, carry):
        z = jnp.zeros((16,), jnp.float32)
        for f in range(D // 16):
            rows0[i, pl.ds(f * 16, 16)] = z
        return carry

    lax.fori_loop(0, C, _zero_row, 0)
    for k in range(RPT // C):
        pltpu.sync_copy(rows0, agg_sh.at[pl.ds(s * RPT + k * C, C)])
    plsc.subcore_barrier()

    def _gdesc(j, buf, sem):
        return pltpu.make_async_copy(
            h_hbm.at[src_all.at[pl.ds(j * C, C)]], buf, sem)

    def _eadesc(j, eabuf, sem):
        return pltpu.make_async_copy(
            ea_hbm.at[pl.ds(ebase + j * C, C)], eabuf, sem)

    def _start(j, buf, eabuf, sem):
        _gdesc(j, buf, sem).start()
        _eadesc(j, eabuf, sem).start()

    def _wait(j, buf, eabuf, sem):
        _gdesc(j, buf, sem).wait()
        _eadesc(j, eabuf, sem).wait()

    def _process(j, buf, eabuf):
        def _grp(g, inner):
            e16 = eabuf[pl.ds(g * 16, 16)]
            for i in range(16):
                r = g * 16 + i
                eb = _lane_bcast(e16, i)
                for f in range(D // 16):
                    buf[r, pl.ds(f * 16, 16)] = (
                        buf[r, pl.ds(f * 16, 16)] * eb)
            return inner

        lax.fori_loop(0, C // 16, _grp, 0)
        for k in range(C // 16):
            dst_v[pl.ds(k * 16, 16)] = dst_all[pl.ds(j * C + k * 16, 16)]
        pltpu.sync_copy(buf, agg_sh.at[dst_v], add=True)

    _start(0, rows0, ea0, sem0)
    _start(1, rows1, ea1, sem1)

    def _pair(i, carry):
        j0 = 2 * i
        _wait(j0, rows0, ea0, sem0)
        _process(j0, rows0, ea0)
        _start(j0 + 2, rows0, ea0, sem0)

        j1 = j0 + 1
        _wait(j1, rows1, ea1, sem1)
        _process(j1, rows1, ea1)

        @pl.when(j1 + 2 < NCHUNK)
        def _():
            _start(j1 + 2, rows1, ea1, sem1)

        return carry

    lax.fori_loop(0, NCHUNK // 2, _pair, 0)
    _wait(NCHUNK - 1, rows0, ea0, sem0)
    _process(NCHUNK - 1, rows0, ea0)

    plsc.subcore_barrier()
    pltpu.sync_copy(agg_sh.at[pl.ds(s * RPT, RPT)],
                    out_hbm.at[c, pl.ds(s * RPT, RPT)])


def _dense_body(a0, a1, h, wr, b, wt, o):
    acc = a0[...] + a1[...]
    o[...] = (jnp.dot(acc, wr[...], preferred_element_type=jnp.float32)
              + b[...]
              + jnp.dot(h[...], wt[...], preferred_element_type=jnp.float32))


def _dense(agg, h, WrT, b, WtT):
    BN = 1000
    return pl.pallas_call(
        _dense_body,
        grid=(N // BN,),
        in_specs=[
            pl.BlockSpec((BN, D), lambda i: (i, 0)),
            pl.BlockSpec((BN, D), lambda i: (i, 0)),
            pl.BlockSpec((BN, D), lambda i: (i, 0)),
            pl.BlockSpec((D, D), lambda i: (0, 0)),
            pl.BlockSpec((1, D), lambda i: (0, 0)),
            pl.BlockSpec((D, D), lambda i: (0, 0)),
        ],
        out_specs=pl.BlockSpec((BN, D), lambda i: (i, 0)),
        out_shape=jax.ShapeDtypeStruct((N, D), jnp.float32),
    )(agg[0], agg[1], h, WrT, b, WtT)


def _readout_body(h, w1, b1, w2, b2, bt, o):
    t = jnp.maximum(
        jnp.dot(h[...], w1[...], preferred_element_type=jnp.float32) + b1[...],
        0.0)
    y = jnp.dot(t, w2[...], preferred_element_type=jnp.float32) + b2[...]
    gids = lax.broadcasted_iota(jnp.int32, (G, N), 0)
    m = (bt[...] == gids).astype(jnp.float32)
    sums = jnp.dot(m, y, preferred_element_type=jnp.float32)
    cnts = jnp.sum(m, axis=1, keepdims=True)
    o[...] = sums / jnp.maximum(cnts, 1.0)


def _readout(h, W1T, b1, W2T, b2, batch2d):
    return pl.pallas_call(
        _readout_body,
        out_shape=jax.ShapeDtypeStruct((G, 1), jnp.float32),
    )(h, W1T, b1, W2T, b2, batch2d)


def kernel(x, edge_index, edge_attr, batch,
           W_rel0, b_rel0, W_root0,
           W_rel1, b_rel1, W_root1,
           W_rel2, b_rel2, W_root2,
           W_ro1, b_ro1, W_ro2, b_ro2):
    src = edge_index[0]
    dst = edge_index[1]
    h = x
    for Wr, b, Wt in ((W_rel0, b_rel0, W_root0),
                      (W_rel1, b_rel1, W_root1),
                      (W_rel2, b_rel2, W_root2)):
        agg = _spmm(h, src, dst, edge_attr)
        h = _dense(agg[:, :N, :], h, Wr.T, b.reshape(1, D), Wt.T)
    return _readout(h, W_ro1.T, b_ro1.reshape(1, D // 2),
                    W_ro2.T, b_ro2.reshape(1, 1), batch.reshape(1, N))
